# baseline (speedup 1.0000x reference)
import functools

import jax
import jax.numpy as jnp
from jax import lax
from jax.experimental import pallas as pl
from jax.experimental.pallas import tpu as pltpu
from jax.experimental.pallas import tpu_sc as plsc

BATCH = 16384
N_ENT = 1000000
D = 64
NC = 2
NS = 16
NW = NC * NS
BPW = BATCH // NW
C = 128
NCH = BPW // C

PACK_W = 8192
PACK_GRID = -(-N_ENT // PACK_W)


def _pack_body(re_ref, im_ref, o_ref):
    o_ref[...] = jnp.concatenate([re_ref[...].T, im_ref[...].T], axis=1)


_pack_call = pl.pallas_call(
    _pack_body,
    grid=(PACK_GRID,),
    in_specs=[
        pl.BlockSpec((D, PACK_W), lambda i: (0, i)),
        pl.BlockSpec((D, PACK_W), lambda i: (0, i)),
    ],
    out_specs=pl.BlockSpec((PACK_W, 2 * D), lambda i: (i, 0)),
    out_shape=jax.ShapeDtypeStruct((N_ENT, 2 * D), jnp.float32),
    compiler_params=pltpu.CompilerParams(fuse_transposed_lhs_in_matmul=True),
)

_mesh = plsc.VectorSubcoreMesh(core_axis_name="c", subcore_axis_name="s")


@functools.partial(
    pl.kernel,
    mesh=_mesh,
    out_type=jax.ShapeDtypeStruct((BATCH // 8, 128), jnp.float32),
    scratch_types=[
        pltpu.VMEM((NCH, C), jnp.int32),
        pltpu.VMEM((NCH, C), jnp.int32),
        pltpu.VMEM((NCH, C), jnp.int32),
        pltpu.VMEM((C, 2 * D), jnp.float32),
        pltpu.VMEM((C, 2 * D), jnp.float32),
        pltpu.VMEM((C, 2 * D), jnp.float32),
        pltpu.VMEM((C // 8, 128), jnp.float32),
        pltpu.SemaphoreType.DMA,
    ],
)
def _complex_partial_kernel(h_hbm, r_hbm, t_hbm, ent_hbm, rel_hbm, out_hbm,
                            hi_v, ri_v, ti_v, hv, tv, rv, pacc_v, sem):
    cid = lax.axis_index("c")
    sid = lax.axis_index("s")
    wid = sid * NC + cid

    pltpu.sync_copy(h_hbm.at[wid], hi_v)
    pltpu.sync_copy(r_hbm.at[wid], ri_v)
    pltpu.sync_copy(t_hbm.at[wid], ti_v)

    for ch in range(NCH):
        cp1 = pltpu.async_copy(ent_hbm.at[hi_v.at[ch]], hv, sem)
        cp2 = pltpu.async_copy(ent_hbm.at[ti_v.at[ch]], tv, sem)
        cp3 = pltpu.async_copy(rel_hbm.at[ri_v.at[ch]], rv, sem)
        cp1.wait()
        cp2.wait()
        cp3.wait()

        def row_body(row, carry):
            acc = jnp.zeros((16,), jnp.float32)
            for j in range(D // 16):
                lo = pl.ds(j * 16, 16)
                hi = pl.ds(D + j * 16, 16)
                a = hv[row, lo]
                b = hv[row, hi]
                c = tv[row, lo]
                d = tv[row, hi]
                p = rv[row, lo]
                q = rv[row, hi]
                acc = acc + p * (a * c + b * d) + q * (a * d - b * c)
            pacc_v[row // 8, pl.ds((row % 8) * 16, 16)] = acc
            return carry

        lax.fori_loop(0, C, row_body, 0)

        pltpu.sync_copy(pacc_v,
                        out_hbm.at[pl.ds(wid * (BPW // 8) + ch * (C // 8),
                                         C // 8)])


def _reduce_body(x_ref, o_ref):
    x = x_ref[...]
    rows = lax.broadcasted_iota(jnp.int32, (128, 128), 0)
    cols = lax.broadcasted_iota(jnp.int32, (128, 128), 1)
    m = (rows // 16 == cols).astype(jnp.float32)
    o_ref[...] = -jnp.dot(x, m, preferred_element_type=jnp.float32)


_reduce_call = pl.pallas_call(
    _reduce_body,
    out_shape=jax.ShapeDtypeStruct((BATCH // 8, 128), jnp.float32),
)


def kernel(h, r, t, entity_re, entity_im, relation_re, relation_im):
    h3 = h.astype(jnp.int32).reshape(NW, NCH, C)
    r3 = r.astype(jnp.int32).reshape(NW, NCH, C)
    t3 = t.astype(jnp.int32).reshape(NW, NCH, C)
    ent = _pack_call(entity_re.T, entity_im.T)
    rel = jnp.concatenate([relation_re, relation_im], axis=1)
    partial = _complex_partial_kernel(h3, r3, t3, ent, rel)
    return _reduce_call(partial)[:, :8].reshape(BATCH)


# speedup vs baseline: 1.2991x; 1.2991x over previous
import functools

import jax
import jax.numpy as jnp
from jax import lax
from jax.experimental import pallas as pl
from jax.experimental.pallas import tpu as pltpu
from jax.experimental.pallas import tpu_sc as plsc

BATCH = 16384
N_ENT = 1000000
D = 64
NC = 2
NS = 16
NW = NC * NS
BPW = BATCH // NW
C = 128
NCH = BPW // C

PACK_W = 8192
PACK_GRID = -(-N_ENT // PACK_W)


def _pack_body(re_ref, im_ref, o_ref):
    x = jnp.concatenate([re_ref[...], im_ref[...]], axis=0)
    o_ref[...] = x.T


_pack_call = pl.pallas_call(
    _pack_body,
    grid=(PACK_GRID,),
    in_specs=[
        pl.BlockSpec((D, PACK_W), lambda i: (0, i)),
        pl.BlockSpec((D, PACK_W), lambda i: (0, i)),
    ],
    out_specs=pl.BlockSpec((PACK_W, 2 * D), lambda i: (i, 0)),
    out_shape=jax.ShapeDtypeStruct((N_ENT, 2 * D), jnp.float32),
    compiler_params=pltpu.CompilerParams(fuse_transposed_lhs_in_matmul=True),
)

_mesh = plsc.VectorSubcoreMesh(core_axis_name="c", subcore_axis_name="s")


@functools.partial(
    pl.kernel,
    mesh=_mesh,
    out_type=jax.ShapeDtypeStruct((BATCH // 8, 128), jnp.float32),
    scratch_types=[
        pltpu.VMEM((NCH, C), jnp.int32),
        pltpu.VMEM((NCH, C), jnp.int32),
        pltpu.VMEM((NCH, C), jnp.int32),
        pltpu.VMEM((C, 2 * D), jnp.float32),
        pltpu.VMEM((C, 2 * D), jnp.float32),
        pltpu.VMEM((C, 2 * D), jnp.float32),
        pltpu.VMEM((C // 8, 128), jnp.float32),
        pltpu.SemaphoreType.DMA,
    ],
)
def _complex_partial_kernel(h_hbm, r_hbm, t_hbm, ent_hbm, rel_hbm, out_hbm,
                            hi_v, ri_v, ti_v, hv, tv, rv, pacc_v, sem):
    cid = lax.axis_index("c")
    sid = lax.axis_index("s")
    wid = sid * NC + cid

    pltpu.sync_copy(h_hbm.at[wid], hi_v)
    pltpu.sync_copy(r_hbm.at[wid], ri_v)
    pltpu.sync_copy(t_hbm.at[wid], ti_v)

    for ch in range(NCH):
        cp1 = pltpu.async_copy(ent_hbm.at[hi_v.at[ch]], hv, sem)
        cp2 = pltpu.async_copy(ent_hbm.at[ti_v.at[ch]], tv, sem)
        cp3 = pltpu.async_copy(rel_hbm.at[ri_v.at[ch]], rv, sem)
        cp1.wait()
        cp2.wait()
        cp3.wait()

        def row_body(row, carry):
            acc = jnp.zeros((16,), jnp.float32)
            for j in range(D // 16):
                lo = pl.ds(j * 16, 16)
                hi = pl.ds(D + j * 16, 16)
                a = hv[row, lo]
                b = hv[row, hi]
                c = tv[row, lo]
                d = tv[row, hi]
                p = rv[row, lo]
                q = rv[row, hi]
                acc = acc + p * (a * c + b * d) + q * (a * d - b * c)
            pacc_v[row // 8, pl.ds((row % 8) * 16, 16)] = acc
            return carry

        lax.fori_loop(0, C, row_body, 0)

        pltpu.sync_copy(pacc_v,
                        out_hbm.at[pl.ds(wid * (BPW // 8) + ch * (C // 8),
                                         C // 8)])


def _reduce_body(x_ref, o_ref):
    x = x_ref[...]
    rows = lax.broadcasted_iota(jnp.int32, (128, 128), 0)
    cols = lax.broadcasted_iota(jnp.int32, (128, 128), 1)
    m = (rows // 16 == cols).astype(jnp.float32)
    o_ref[...] = -jnp.dot(x, m, preferred_element_type=jnp.float32)


_reduce_call = pl.pallas_call(
    _reduce_body,
    out_shape=jax.ShapeDtypeStruct((BATCH // 8, 128), jnp.float32),
)


def kernel(h, r, t, entity_re, entity_im, relation_re, relation_im):
    h3 = h.astype(jnp.int32).reshape(NW, NCH, C)
    r3 = r.astype(jnp.int32).reshape(NW, NCH, C)
    t3 = t.astype(jnp.int32).reshape(NW, NCH, C)
    ent = _pack_call(entity_re.T, entity_im.T)
    rel = jnp.concatenate([relation_re, relation_im], axis=1)
    partial = _complex_partial_kernel(h3, r3, t3, ent, rel)
    return _reduce_call(partial)[:, :8].reshape(BATCH)


# speedup vs baseline: 1.3295x; 1.0234x over previous
import functools

import jax
import jax.numpy as jnp
from jax import lax
from jax.experimental import pallas as pl
from jax.experimental.pallas import tpu as pltpu
from jax.experimental.pallas import tpu_sc as plsc

BATCH = 16384
N_ENT = 1000000
D = 64
NC = 2
NS = 16
NW = NC * NS
BPW = BATCH // NW
C = 128
NCH = BPW // C

PACK_W = 16384
PACK_GRID = -(-N_ENT // PACK_W)


def _pack_body(re_ref, im_ref, o_ref):
    x = jnp.concatenate([re_ref[...], im_ref[...]], axis=0)
    o_ref[...] = x.T


_pack_call = pl.pallas_call(
    _pack_body,
    grid=(PACK_GRID,),
    in_specs=[
        pl.BlockSpec((D, PACK_W), lambda i: (0, i)),
        pl.BlockSpec((D, PACK_W), lambda i: (0, i)),
    ],
    out_specs=pl.BlockSpec((PACK_W, 2 * D), lambda i: (i, 0)),
    out_shape=jax.ShapeDtypeStruct((N_ENT, 2 * D), jnp.float32),
    compiler_params=pltpu.CompilerParams(fuse_transposed_lhs_in_matmul=True),
)

_mesh = plsc.VectorSubcoreMesh(core_axis_name="c", subcore_axis_name="s")


@functools.partial(
    pl.kernel,
    mesh=_mesh,
    out_type=jax.ShapeDtypeStruct((BATCH // 8, 128), jnp.float32),
    scratch_types=[
        pltpu.VMEM((NCH, C), jnp.int32),
        pltpu.VMEM((NCH, C), jnp.int32),
        pltpu.VMEM((NCH, C), jnp.int32),
        pltpu.VMEM((C, 2 * D), jnp.float32),
        pltpu.VMEM((C, 2 * D), jnp.float32),
        pltpu.VMEM((C, 2 * D), jnp.float32),
        pltpu.VMEM((C // 8, 128), jnp.float32),
        pltpu.SemaphoreType.DMA,
    ],
)
def _complex_partial_kernel(h_hbm, r_hbm, t_hbm, ent_hbm, rel_hbm, out_hbm,
                            hi_v, ri_v, ti_v, hv, tv, rv, pacc_v, sem):
    cid = lax.axis_index("c")
    sid = lax.axis_index("s")
    wid = sid * NC + cid

    pltpu.sync_copy(h_hbm.at[wid], hi_v)
    pltpu.sync_copy(r_hbm.at[wid], ri_v)
    pltpu.sync_copy(t_hbm.at[wid], ti_v)

    for ch in range(NCH):
        cp1 = pltpu.async_copy(ent_hbm.at[hi_v.at[ch]], hv, sem)
        cp2 = pltpu.async_copy(ent_hbm.at[ti_v.at[ch]], tv, sem)
        cp3 = pltpu.async_copy(rel_hbm.at[ri_v.at[ch]], rv, sem)
        cp1.wait()
        cp2.wait()
        cp3.wait()

        def row_body(row, carry):
            acc = jnp.zeros((16,), jnp.float32)
            for j in range(D // 16):
                lo = pl.ds(j * 16, 16)
                hi = pl.ds(D + j * 16, 16)
                a = hv[row, lo]
                b = hv[row, hi]
                c = tv[row, lo]
                d = tv[row, hi]
                p = rv[row, lo]
                q = rv[row, hi]
                acc = acc + p * (a * c + b * d) + q * (a * d - b * c)
            pacc_v[row // 8, pl.ds((row % 8) * 16, 16)] = acc
            return carry

        lax.fori_loop(0, C, row_body, 0)

        pltpu.sync_copy(pacc_v,
                        out_hbm.at[pl.ds(wid * (BPW // 8) + ch * (C // 8),
                                         C // 8)])


def _reduce_body(x_ref, o_ref):
    x = x_ref[...]
    rows = lax.broadcasted_iota(jnp.int32, (128, 128), 0)
    cols = lax.broadcasted_iota(jnp.int32, (128, 128), 1)
    m = (rows // 16 == cols).astype(jnp.float32)
    o_ref[...] = -jnp.dot(x, m, preferred_element_type=jnp.float32)


_reduce_call = pl.pallas_call(
    _reduce_body,
    out_shape=jax.ShapeDtypeStruct((BATCH // 8, 128), jnp.float32),
)


def kernel(h, r, t, entity_re, entity_im, relation_re, relation_im):
    h3 = h.astype(jnp.int32).reshape(NW, NCH, C)
    r3 = r.astype(jnp.int32).reshape(NW, NCH, C)
    t3 = t.astype(jnp.int32).reshape(NW, NCH, C)
    ent = _pack_call(entity_re.T, entity_im.T)
    rel = jnp.concatenate([relation_re, relation_im], axis=1)
    partial = _complex_partial_kernel(h3, r3, t3, ent, rel)
    return _reduce_call(partial)[:, :8].reshape(BATCH)
